# baseline (speedup 1.0000x reference)
import functools

import jax
import jax.numpy as jnp
from jax import lax
from jax.experimental import pallas as pl
from jax.experimental.pallas import tpu as pltpu
from jax.experimental.pallas import tpu_sc as plsc

N = 10000
E = 320000
NC, NS = 2, 16
NW = NC * NS
CHUNK = 128
EPW = 10240
EPAD = NW * EPW
NIT = EPW // CHUNK
BLK = 16
CPS = 160
assert CPS % BLK == 0 and NS * CPS * CHUNK == EPAD
ACC_ROWS = 10240
ZR = 64
ROWS_WB = N // NS

_MESH = plsc.VectorSubcoreMesh(
    core_axis_name="c", subcore_axis_name="s", num_cores=NC, num_subcores=NS
)


def _zero_vmem_2d(ref, rows, d):
    cols = d // 16

    def body(t, _):
        ref[t // cols, pl.ds((t % cols) * 16, 16)] = jnp.zeros((16,), jnp.float32)
        return 0

    lax.fori_loop(0, rows * cols, body, 0)


def _zero_vmem_1d(ref, n):
    def body(t, _):
        ref[pl.ds(t * 16, 16)] = jnp.zeros((16,), jnp.float32)
        return 0

    lax.fori_loop(0, n // 16, body, 0)


@functools.partial(
    pl.kernel,
    out_type=jax.ShapeDtypeStruct((NC * N,), jnp.float32),
    mesh=_MESH,
    scratch_types=[
        pltpu.VMEM((NIT, CHUNK), jnp.int32),
        pltpu.VMEM((CHUNK,), jnp.float32),
        pltpu.VMEM((640,), jnp.float32),
        pltpu.VMEM((1000,), jnp.float32),
        pltpu.VMEM_SHARED((ACC_ROWS,), jnp.float32),
    ],
)
def _deg_kernel(dst_hbm, out_hbm, didxb, ones, ztile, wbuf, acc):
    c = lax.axis_index("c")
    s = lax.axis_index("s")
    wid = c * NS + s

    _zero_vmem_1d(ztile, 640)

    def fill_ones(t, _):
        ones[pl.ds(t * 16, 16)] = jnp.ones((16,), jnp.float32)
        return 0

    lax.fori_loop(0, CHUNK // 16, fill_ones, 0)

    pltpu.sync_copy(ztile, acc.at[pl.ds(s * 640, 640)])
    plsc.subcore_barrier()

    pltpu.sync_copy(dst_hbm.at[pl.ds(wid * NIT, NIT)], didxb)

    def body(t, _):
        pltpu.sync_copy(ones, acc.at[didxb.at[t]], add=True)
        return 0

    lax.fori_loop(0, NIT, body, 0)
    plsc.subcore_barrier()

    @pl.when(s < 10)
    def _():
        pltpu.sync_copy(acc.at[pl.ds(s * 1000, 1000)], wbuf)
        pltpu.sync_copy(wbuf, out_hbm.at[pl.ds(c * N + s * 1000, 1000)])


def _make_agg(d):
    @functools.partial(
        pl.kernel,
        out_type=jax.ShapeDtypeStruct((N, d), jnp.float32),
        mesh=_MESH,
        scratch_types=[
            pltpu.VMEM((BLK, CHUNK), jnp.int32),
            pltpu.VMEM((BLK, CHUNK), jnp.int32),
            pltpu.VMEM((CHUNK, d), jnp.float32),
            pltpu.VMEM((CHUNK, d), jnp.float32),
            pltpu.VMEM_SHARED((ACC_ROWS, d), jnp.float32),
            pltpu.SemaphoreType.DMA,
            pltpu.SemaphoreType.DMA,
        ],
    )
    def agg(src_hbm, dst_hbm, h_hbm, out_hbm, sidxb, didxb, rowsa, rowsb,
            acc, sema, semb):
        c = lax.axis_index("c")
        s = lax.axis_index("s")

        @pl.when(c == 0)
        def _():
            _zero_vmem_2d(rowsa, CHUNK, d)
            for j in range(ACC_ROWS // NS // CHUNK):
                pltpu.sync_copy(rowsa, acc.at[pl.ds(s * 640 + j * CHUNK, CHUNK)])
            plsc.subcore_barrier()

            def block(bi, _):
                r0 = s * CPS + bi * BLK
                pltpu.sync_copy(src_hbm.at[pl.ds(r0, BLK)], sidxb)
                pltpu.sync_copy(dst_hbm.at[pl.ds(r0, BLK)], didxb)
                pltpu.async_copy(h_hbm.at[sidxb.at[0]], rowsa, sema)

                def pair(g, _):
                    t = 2 * g
                    pltpu.async_copy(h_hbm.at[sidxb.at[t + 1]], rowsb, semb)
                    pltpu.make_async_copy(
                        h_hbm.at[sidxb.at[t]], rowsa, sema
                    ).wait()
                    pltpu.sync_copy(rowsa, acc.at[didxb.at[t]], add=True)

                    @pl.when(t + 2 < BLK)
                    def _():
                        pltpu.async_copy(h_hbm.at[sidxb.at[t + 2]], rowsa, sema)

                    pltpu.make_async_copy(
                        h_hbm.at[sidxb.at[t + 1]], rowsb, semb
                    ).wait()
                    pltpu.sync_copy(rowsb, acc.at[didxb.at[t + 1]], add=True)
                    return 0

                lax.fori_loop(0, BLK // 2, pair, 0)
                return 0

            lax.fori_loop(0, CPS // BLK, block, 0)
            plsc.subcore_barrier()
            rows = rowsa

            @pl.when(s < 15)
            def _():
                for j in range(5):
                    sz = 120 if j == 4 else CHUNK
                    r0 = s * 632 + j * CHUNK
                    pltpu.sync_copy(acc.at[pl.ds(r0, sz)], rows.at[pl.ds(0, sz)])
                    pltpu.sync_copy(rows.at[pl.ds(0, sz)], out_hbm.at[pl.ds(r0, sz)])

            @pl.when(s == 15)
            def _():
                for j in range(5):
                    sz = 8 if j == 4 else CHUNK
                    r0 = 15 * 632 + j * CHUNK
                    pltpu.sync_copy(acc.at[pl.ds(r0, sz)], rows.at[pl.ds(0, sz)])
                    pltpu.sync_copy(rows.at[pl.ds(0, sz)], out_hbm.at[pl.ds(r0, sz)])

    return agg


_agg128 = _make_agg(128)


_GRID = 10
_BR = N // _GRID


def _h1_body(x_ref, w_ref, d0_ref, d1_ref, o_ref, dinv_ref):
    dinv = lax.rsqrt(d0_ref[...] + d1_ref[...] + 1.0)
    dinv_ref[...] = dinv
    h = jnp.dot(x_ref[...], w_ref[...], preferred_element_type=jnp.float32)
    o_ref[...] = dinv * h


def _h1(x, w, d0, d1):
    return pl.pallas_call(
        _h1_body,
        grid=(_GRID,),
        in_specs=[
            pl.BlockSpec((_BR, 128), lambda i: (i, 0)),
            pl.BlockSpec((128, 128), lambda i: (0, 0)),
            pl.BlockSpec((_BR, 1), lambda i: (i, 0)),
            pl.BlockSpec((_BR, 1), lambda i: (i, 0)),
        ],
        out_specs=[
            pl.BlockSpec((_BR, 128), lambda i: (i, 0)),
            pl.BlockSpec((_BR, 1), lambda i: (i, 0)),
        ],
        out_shape=[
            jax.ShapeDtypeStruct((N, 128), jnp.float32),
            jax.ShapeDtypeStruct((N, 1), jnp.float32),
        ],
    )(x, w, d0, d1)


def _fuse_body(agg_ref, ht_ref, dinv_ref, b_ref, w_ref, o_ref):
    dinv = dinv_ref[...]
    pre = dinv * (agg_ref[...] + ht_ref[...]) + b_ref[...]
    x = jnp.maximum(pre, 0.0)
    o_ref[...] = dinv * jnp.dot(
        x, w_ref[...], preferred_element_type=jnp.float32
    )


def _fuse(agg, ht, dinv, b, w, din, dout):
    return pl.pallas_call(
        _fuse_body,
        grid=(_GRID,),
        in_specs=[
            pl.BlockSpec((_BR, din), lambda i: (i, 0)),
            pl.BlockSpec((_BR, din), lambda i: (i, 0)),
            pl.BlockSpec((_BR, 1), lambda i: (i, 0)),
            pl.BlockSpec((1, din), lambda i: (0, 0)),
            pl.BlockSpec((din, dout), lambda i: (0, 0)),
        ],
        out_specs=pl.BlockSpec((_BR, dout), lambda i: (i, 0)),
        out_shape=jax.ShapeDtypeStruct((N, dout), jnp.float32),
    )(agg, ht, dinv, b, w)


def _final_body(agg_ref, ht_ref, dinv_ref, b_ref, o_ref):
    dinv = dinv_ref[...]
    tot = agg_ref[...] + ht_ref[...]
    o_ref[...] = dinv * tot[:, :64] + b_ref[...]


def _final(agg, ht, dinv, b):
    return pl.pallas_call(
        _final_body,
        grid=(_GRID,),
        in_specs=[
            pl.BlockSpec((_BR, 128), lambda i: (i, 0)),
            pl.BlockSpec((_BR, 128), lambda i: (i, 0)),
            pl.BlockSpec((_BR, 1), lambda i: (i, 0)),
            pl.BlockSpec((1, 64), lambda i: (0, 0)),
        ],
        out_specs=pl.BlockSpec((_BR, 64), lambda i: (i, 0)),
        out_shape=jax.ShapeDtypeStruct((N, 64), jnp.float32),
    )(agg, ht, dinv, b)


def kernel(features, edge_index, W1, b1, W2, b2, W3, b3):
    pad = EPAD - E
    src = jnp.concatenate(
        [edge_index[0].astype(jnp.int32), jnp.zeros((pad,), jnp.int32)]
    ).reshape(EPAD // CHUNK, CHUNK)
    dst = jnp.concatenate(
        [edge_index[1].astype(jnp.int32), jnp.full((pad,), N, jnp.int32)]
    ).reshape(EPAD // CHUNK, CHUNK)
    degp = _deg_kernel(dst).reshape(NC, N)
    ht1, dinv = _h1(features, W1, degp[0][:, None], degp[1][:, None])
    agg1 = _agg128(src, dst, ht1)
    ht2 = _fuse(agg1, ht1, dinv, b1[None, :], W2, 128, 128)
    agg2 = _agg128(src, dst, ht2)
    w3p = jnp.pad(W3, ((0, 0), (0, 64)))
    ht3 = _fuse(agg2, ht2, dinv, b2[None, :], w3p, 128, 128)
    agg3 = _agg128(src, dst, ht3)
    return _final(agg3, ht3, dinv, b3[None, :])


# speedup vs baseline: 1.1713x; 1.1713x over previous
import functools

import jax
import jax.numpy as jnp
from jax import lax
from jax.experimental import pallas as pl
from jax.experimental.pallas import tpu as pltpu
from jax.experimental.pallas import tpu_sc as plsc

N = 10000
E = 320000
NC, NS = 2, 16
NW = NC * NS
CHUNK = 128
EPW = 10240
EPAD = NW * EPW
NIT = EPW // CHUNK
BLK = 16
CPS0 = 96
CPS1 = 64
assert CPS0 % BLK == 0 and CPS1 % BLK == 0 and CPS0 + CPS1 == 2 * NIT
ACC_ROWS = 10240
ZR = 64
ROWS_WB = N // NS

_MESH = plsc.VectorSubcoreMesh(
    core_axis_name="c", subcore_axis_name="s", num_cores=NC, num_subcores=NS
)


def _zero_vmem_2d(ref, rows, d):
    cols = d // 16

    def body(t, _):
        ref[t // cols, pl.ds((t % cols) * 16, 16)] = jnp.zeros((16,), jnp.float32)
        return 0

    lax.fori_loop(0, rows * cols, body, 0)


def _zero_vmem_1d(ref, n):
    def body(t, _):
        ref[pl.ds(t * 16, 16)] = jnp.zeros((16,), jnp.float32)
        return 0

    lax.fori_loop(0, n // 16, body, 0)


@functools.partial(
    pl.kernel,
    out_type=jax.ShapeDtypeStruct((NC * N,), jnp.float32),
    mesh=_MESH,
    scratch_types=[
        pltpu.VMEM((NIT, CHUNK), jnp.int32),
        pltpu.VMEM((CHUNK,), jnp.float32),
        pltpu.VMEM((640,), jnp.float32),
        pltpu.VMEM((1000,), jnp.float32),
        pltpu.VMEM_SHARED((ACC_ROWS,), jnp.float32),
    ],
)
def _deg_kernel(dst_hbm, out_hbm, didxb, ones, ztile, wbuf, acc):
    c = lax.axis_index("c")
    s = lax.axis_index("s")
    wid = c * NS + s

    _zero_vmem_1d(ztile, 640)

    def fill_ones(t, _):
        ones[pl.ds(t * 16, 16)] = jnp.ones((16,), jnp.float32)
        return 0

    lax.fori_loop(0, CHUNK // 16, fill_ones, 0)

    pltpu.sync_copy(ztile, acc.at[pl.ds(s * 640, 640)])
    plsc.subcore_barrier()

    pltpu.sync_copy(dst_hbm.at[pl.ds(wid * NIT, NIT)], didxb)

    def body(t, _):
        pltpu.sync_copy(ones, acc.at[didxb.at[t]], add=True)
        return 0

    lax.fori_loop(0, NIT, body, 0)
    plsc.subcore_barrier()

    @pl.when(s < 10)
    def _():
        pltpu.sync_copy(acc.at[pl.ds(s * 1000, 1000)], wbuf)
        pltpu.sync_copy(wbuf, out_hbm.at[pl.ds(c * N + s * 1000, 1000)])


def _make_agg(d):
    @functools.partial(
        pl.kernel,
        out_type=jax.ShapeDtypeStruct((NC, N, d), jnp.float32),
        mesh=_MESH,
        scratch_types=[
            pltpu.VMEM((BLK, CHUNK), jnp.int32),
            pltpu.VMEM((BLK, CHUNK), jnp.int32),
            pltpu.VMEM((CHUNK, d), jnp.float32),
            pltpu.VMEM((CHUNK, d), jnp.float32),
            pltpu.VMEM_SHARED((ACC_ROWS, d), jnp.float32),
            pltpu.SemaphoreType.DMA,
            pltpu.SemaphoreType.DMA,
        ],
    )
    def agg(src_hbm, dst_hbm, h_hbm, out_hbm, sidxb, didxb, rowsa, rowsb,
            acc, sema, semb):
        c = lax.axis_index("c")
        s = lax.axis_index("s")

        _zero_vmem_2d(rowsa, CHUNK, d)
        for j in range(ACC_ROWS // NS // CHUNK):
            pltpu.sync_copy(rowsa, acc.at[pl.ds(s * 640 + j * CHUNK, CHUNK)])
        plsc.subcore_barrier()

        nblk = lax.select(c == 0, CPS0 // BLK, CPS1 // BLK)
        base = lax.select(c == 0, s * CPS0, NS * CPS0 + s * CPS1)

        def block(bi, _):
            r0 = base + bi * BLK
            pltpu.sync_copy(src_hbm.at[pl.ds(r0, BLK)], sidxb)
            pltpu.sync_copy(dst_hbm.at[pl.ds(r0, BLK)], didxb)
            pltpu.async_copy(h_hbm.at[sidxb.at[0]], rowsa, sema)

            def pair(g, _):
                t = 2 * g
                pltpu.async_copy(h_hbm.at[sidxb.at[t + 1]], rowsb, semb)
                pltpu.make_async_copy(h_hbm.at[sidxb.at[t]], rowsa, sema).wait()
                pltpu.sync_copy(rowsa, acc.at[didxb.at[t]], add=True)

                @pl.when(t + 2 < BLK)
                def _():
                    pltpu.async_copy(h_hbm.at[sidxb.at[t + 2]], rowsa, sema)

                pltpu.make_async_copy(
                    h_hbm.at[sidxb.at[t + 1]], rowsb, semb
                ).wait()
                pltpu.sync_copy(rowsb, acc.at[didxb.at[t + 1]], add=True)
                return 0

            lax.fori_loop(0, BLK // 2, pair, 0)
            return 0

        lax.fori_loop(0, nblk, block, 0)
        plsc.subcore_barrier()
        rows = rowsa

        @pl.when(s < 15)
        def _():
            for j in range(5):
                sz = 120 if j == 4 else CHUNK
                r0 = s * 632 + j * CHUNK
                pltpu.sync_copy(acc.at[pl.ds(r0, sz)], rows.at[pl.ds(0, sz)])
                pltpu.sync_copy(rows.at[pl.ds(0, sz)], out_hbm.at[c, pl.ds(r0, sz)])

        @pl.when(s == 15)
        def _():
            for j in range(5):
                sz = 8 if j == 4 else CHUNK
                r0 = 15 * 632 + j * CHUNK
                pltpu.sync_copy(acc.at[pl.ds(r0, sz)], rows.at[pl.ds(0, sz)])
                pltpu.sync_copy(rows.at[pl.ds(0, sz)], out_hbm.at[c, pl.ds(r0, sz)])

    return agg


_agg128 = _make_agg(128)


_GRID = 10
_BR = N // _GRID


def _h1_body(x_ref, w_ref, d0_ref, d1_ref, o_ref, dinv_ref):
    dinv = lax.rsqrt(d0_ref[...] + d1_ref[...] + 1.0)
    dinv_ref[...] = dinv
    h = jnp.dot(x_ref[...], w_ref[...], preferred_element_type=jnp.float32)
    o_ref[...] = dinv * h


def _h1(x, w, d0, d1):
    return pl.pallas_call(
        _h1_body,
        grid=(_GRID,),
        in_specs=[
            pl.BlockSpec((_BR, 128), lambda i: (i, 0)),
            pl.BlockSpec((128, 128), lambda i: (0, 0)),
            pl.BlockSpec((_BR, 1), lambda i: (i, 0)),
            pl.BlockSpec((_BR, 1), lambda i: (i, 0)),
        ],
        out_specs=[
            pl.BlockSpec((_BR, 128), lambda i: (i, 0)),
            pl.BlockSpec((_BR, 1), lambda i: (i, 0)),
        ],
        out_shape=[
            jax.ShapeDtypeStruct((N, 128), jnp.float32),
            jax.ShapeDtypeStruct((N, 1), jnp.float32),
        ],
    )(x, w, d0, d1)


def _fuse_body(aggp_ref, ht_ref, dinv_ref, b_ref, w_ref, o_ref):
    dinv = dinv_ref[...]
    pre = dinv * (aggp_ref[0] + aggp_ref[1] + ht_ref[...]) + b_ref[...]
    x = jnp.maximum(pre, 0.0)
    o_ref[...] = dinv * jnp.dot(
        x, w_ref[...], preferred_element_type=jnp.float32
    )


def _fuse(aggp, ht, dinv, b, w, din, dout):
    return pl.pallas_call(
        _fuse_body,
        grid=(_GRID,),
        in_specs=[
            pl.BlockSpec((NC, _BR, din), lambda i: (0, i, 0)),
            pl.BlockSpec((_BR, din), lambda i: (i, 0)),
            pl.BlockSpec((_BR, 1), lambda i: (i, 0)),
            pl.BlockSpec((1, din), lambda i: (0, 0)),
            pl.BlockSpec((din, dout), lambda i: (0, 0)),
        ],
        out_specs=pl.BlockSpec((_BR, dout), lambda i: (i, 0)),
        out_shape=jax.ShapeDtypeStruct((N, dout), jnp.float32),
    )(aggp, ht, dinv, b, w)


def _final_body(aggp_ref, ht_ref, dinv_ref, b_ref, o_ref):
    dinv = dinv_ref[...]
    tot = aggp_ref[0] + aggp_ref[1] + ht_ref[...]
    o_ref[...] = dinv * tot[:, :64] + b_ref[...]


def _final(aggp, ht, dinv, b):
    return pl.pallas_call(
        _final_body,
        grid=(_GRID,),
        in_specs=[
            pl.BlockSpec((NC, _BR, 128), lambda i: (0, i, 0)),
            pl.BlockSpec((_BR, 128), lambda i: (i, 0)),
            pl.BlockSpec((_BR, 1), lambda i: (i, 0)),
            pl.BlockSpec((1, 64), lambda i: (0, 0)),
        ],
        out_specs=pl.BlockSpec((_BR, 64), lambda i: (i, 0)),
        out_shape=jax.ShapeDtypeStruct((N, 64), jnp.float32),
    )(aggp, ht, dinv, b)


def kernel(features, edge_index, W1, b1, W2, b2, W3, b3):
    pad = EPAD - E
    src = jnp.concatenate(
        [edge_index[0].astype(jnp.int32), jnp.zeros((pad,), jnp.int32)]
    ).reshape(EPAD // CHUNK, CHUNK)
    dst = jnp.concatenate(
        [edge_index[1].astype(jnp.int32), jnp.full((pad,), N, jnp.int32)]
    ).reshape(EPAD // CHUNK, CHUNK)
    degp = _deg_kernel(dst).reshape(NC, N)
    ht1, dinv = _h1(features, W1, degp[0][:, None], degp[1][:, None])
    agg1 = _agg128(src, dst, ht1)
    ht2 = _fuse(agg1, ht1, dinv, b1[None, :], W2, 128, 128)
    agg2 = _agg128(src, dst, ht2)
    w3p = jnp.pad(W3, ((0, 0), (0, 64)))
    ht3 = _fuse(agg2, ht2, dinv, b2[None, :], w3p, 128, 128)
    agg3 = _agg128(src, dst, ht3)
    return _final(agg3, ht3, dinv, b3[None, :])


# speedup vs baseline: 1.1931x; 1.0186x over previous
import functools

import jax
import jax.numpy as jnp
from jax import lax
from jax.experimental import pallas as pl
from jax.experimental.pallas import tpu as pltpu
from jax.experimental.pallas import tpu_sc as plsc

N = 10000
E = 320000
NC, NS = 2, 16
NW = NC * NS
CHUNK = 128
EPW = 10240
EPAD = NW * EPW
NIT = EPW // CHUNK
BLK = 16
CPS0 = 112
CPS1 = 48
assert CPS0 % BLK == 0 and CPS1 % BLK == 0 and CPS0 + CPS1 == 2 * NIT
ACC_ROWS = 10240
ZR = 64
ROWS_WB = N // NS

_MESH = plsc.VectorSubcoreMesh(
    core_axis_name="c", subcore_axis_name="s", num_cores=NC, num_subcores=NS
)


def _zero_vmem_2d(ref, rows, d):
    cols = d // 16

    def body(t, _):
        ref[t // cols, pl.ds((t % cols) * 16, 16)] = jnp.zeros((16,), jnp.float32)
        return 0

    lax.fori_loop(0, rows * cols, body, 0)


def _zero_vmem_1d(ref, n):
    def body(t, _):
        ref[pl.ds(t * 16, 16)] = jnp.zeros((16,), jnp.float32)
        return 0

    lax.fori_loop(0, n // 16, body, 0)


@functools.partial(
    pl.kernel,
    out_type=jax.ShapeDtypeStruct((NC * N,), jnp.float32),
    mesh=_MESH,
    scratch_types=[
        pltpu.VMEM((NIT, CHUNK), jnp.int32),
        pltpu.VMEM((CHUNK,), jnp.float32),
        pltpu.VMEM((640,), jnp.float32),
        pltpu.VMEM((1000,), jnp.float32),
        pltpu.VMEM_SHARED((ACC_ROWS,), jnp.float32),
    ],
)
def _deg_kernel(dst_hbm, out_hbm, didxb, ones, ztile, wbuf, acc):
    c = lax.axis_index("c")
    s = lax.axis_index("s")
    wid = c * NS + s

    _zero_vmem_1d(ztile, 640)

    def fill_ones(t, _):
        ones[pl.ds(t * 16, 16)] = jnp.ones((16,), jnp.float32)
        return 0

    lax.fori_loop(0, CHUNK // 16, fill_ones, 0)

    pltpu.sync_copy(ztile, acc.at[pl.ds(s * 640, 640)])
    plsc.subcore_barrier()

    pltpu.sync_copy(dst_hbm.at[pl.ds(wid * NIT, NIT)], didxb)

    def body(t, _):
        pltpu.sync_copy(ones, acc.at[didxb.at[t]], add=True)
        return 0

    lax.fori_loop(0, NIT, body, 0)
    plsc.subcore_barrier()

    @pl.when(s < 10)
    def _():
        pltpu.sync_copy(acc.at[pl.ds(s * 1000, 1000)], wbuf)
        pltpu.sync_copy(wbuf, out_hbm.at[pl.ds(c * N + s * 1000, 1000)])


def _make_agg(d):
    @functools.partial(
        pl.kernel,
        out_type=jax.ShapeDtypeStruct((NC, N, d), jnp.float32),
        mesh=_MESH,
        scratch_types=[
            pltpu.VMEM((BLK, CHUNK), jnp.int32),
            pltpu.VMEM((BLK, CHUNK), jnp.int32),
            pltpu.VMEM((CHUNK, d), jnp.float32),
            pltpu.VMEM((CHUNK, d), jnp.float32),
            pltpu.VMEM_SHARED((ACC_ROWS, d), jnp.float32),
            pltpu.SemaphoreType.DMA,
            pltpu.SemaphoreType.DMA,
        ],
    )
    def agg(src_hbm, dst_hbm, h_hbm, out_hbm, sidxb, didxb, rowsa, rowsb,
            acc, sema, semb):
        c = lax.axis_index("c")
        s = lax.axis_index("s")

        _zero_vmem_2d(rowsa, CHUNK, d)
        for j in range(ACC_ROWS // NS // CHUNK):
            pltpu.sync_copy(rowsa, acc.at[pl.ds(s * 640 + j * CHUNK, CHUNK)])
        plsc.subcore_barrier()

        nblk = lax.select(c == 0, CPS0 // BLK, CPS1 // BLK)
        base = lax.select(c == 0, s * CPS0, NS * CPS0 + s * CPS1)

        def block(bi, _):
            r0 = base + bi * BLK
            pltpu.sync_copy(src_hbm.at[pl.ds(r0, BLK)], sidxb)
            pltpu.sync_copy(dst_hbm.at[pl.ds(r0, BLK)], didxb)
            pltpu.async_copy(h_hbm.at[sidxb.at[0]], rowsa, sema)

            def pair(g, _):
                t = 2 * g
                pltpu.async_copy(h_hbm.at[sidxb.at[t + 1]], rowsb, semb)
                pltpu.make_async_copy(h_hbm.at[sidxb.at[t]], rowsa, sema).wait()
                pltpu.sync_copy(rowsa, acc.at[didxb.at[t]], add=True)

                @pl.when(t + 2 < BLK)
                def _():
                    pltpu.async_copy(h_hbm.at[sidxb.at[t + 2]], rowsa, sema)

                pltpu.make_async_copy(
                    h_hbm.at[sidxb.at[t + 1]], rowsb, semb
                ).wait()
                pltpu.sync_copy(rowsb, acc.at[didxb.at[t + 1]], add=True)
                return 0

            lax.fori_loop(0, BLK // 2, pair, 0)
            return 0

        lax.fori_loop(0, nblk, block, 0)
        plsc.subcore_barrier()
        rows = rowsa

        @pl.when(s < 15)
        def _():
            for j in range(5):
                sz = 120 if j == 4 else CHUNK
                r0 = s * 632 + j * CHUNK
                pltpu.sync_copy(acc.at[pl.ds(r0, sz)], rows.at[pl.ds(0, sz)])
                pltpu.sync_copy(rows.at[pl.ds(0, sz)], out_hbm.at[c, pl.ds(r0, sz)])

        @pl.when(s == 15)
        def _():
            for j in range(5):
                sz = 8 if j == 4 else CHUNK
                r0 = 15 * 632 + j * CHUNK
                pltpu.sync_copy(acc.at[pl.ds(r0, sz)], rows.at[pl.ds(0, sz)])
                pltpu.sync_copy(rows.at[pl.ds(0, sz)], out_hbm.at[c, pl.ds(r0, sz)])

    return agg


_agg128 = _make_agg(128)


_GRID = 10
_BR = N // _GRID


def _h1_body(x_ref, w_ref, d0_ref, d1_ref, o_ref, dinv_ref):
    dinv = lax.rsqrt(d0_ref[...] + d1_ref[...] + 1.0)
    dinv_ref[...] = dinv
    h = jnp.dot(x_ref[...], w_ref[...], preferred_element_type=jnp.float32)
    o_ref[...] = dinv * h


def _h1(x, w, d0, d1):
    return pl.pallas_call(
        _h1_body,
        grid=(_GRID,),
        in_specs=[
            pl.BlockSpec((_BR, 128), lambda i: (i, 0)),
            pl.BlockSpec((128, 128), lambda i: (0, 0)),
            pl.BlockSpec((_BR, 1), lambda i: (i, 0)),
            pl.BlockSpec((_BR, 1), lambda i: (i, 0)),
        ],
        out_specs=[
            pl.BlockSpec((_BR, 128), lambda i: (i, 0)),
            pl.BlockSpec((_BR, 1), lambda i: (i, 0)),
        ],
        out_shape=[
            jax.ShapeDtypeStruct((N, 128), jnp.float32),
            jax.ShapeDtypeStruct((N, 1), jnp.float32),
        ],
    )(x, w, d0, d1)


def _fuse_body(aggp_ref, ht_ref, dinv_ref, b_ref, w_ref, o_ref):
    dinv = dinv_ref[...]
    pre = dinv * (aggp_ref[0] + aggp_ref[1] + ht_ref[...]) + b_ref[...]
    x = jnp.maximum(pre, 0.0)
    o_ref[...] = dinv * jnp.dot(
        x, w_ref[...], preferred_element_type=jnp.float32
    )


def _fuse(aggp, ht, dinv, b, w, din, dout):
    return pl.pallas_call(
        _fuse_body,
        grid=(_GRID,),
        in_specs=[
            pl.BlockSpec((NC, _BR, din), lambda i: (0, i, 0)),
            pl.BlockSpec((_BR, din), lambda i: (i, 0)),
            pl.BlockSpec((_BR, 1), lambda i: (i, 0)),
            pl.BlockSpec((1, din), lambda i: (0, 0)),
            pl.BlockSpec((din, dout), lambda i: (0, 0)),
        ],
        out_specs=pl.BlockSpec((_BR, dout), lambda i: (i, 0)),
        out_shape=jax.ShapeDtypeStruct((N, dout), jnp.float32),
    )(aggp, ht, dinv, b, w)


def _final_body(aggp_ref, ht_ref, dinv_ref, b_ref, o_ref):
    dinv = dinv_ref[...]
    tot = aggp_ref[0] + aggp_ref[1] + ht_ref[...]
    o_ref[...] = dinv * tot[:, :64] + b_ref[...]


def _final(aggp, ht, dinv, b):
    return pl.pallas_call(
        _final_body,
        grid=(_GRID,),
        in_specs=[
            pl.BlockSpec((NC, _BR, 128), lambda i: (0, i, 0)),
            pl.BlockSpec((_BR, 128), lambda i: (i, 0)),
            pl.BlockSpec((_BR, 1), lambda i: (i, 0)),
            pl.BlockSpec((1, 64), lambda i: (0, 0)),
        ],
        out_specs=pl.BlockSpec((_BR, 64), lambda i: (i, 0)),
        out_shape=jax.ShapeDtypeStruct((N, 64), jnp.float32),
    )(aggp, ht, dinv, b)


def kernel(features, edge_index, W1, b1, W2, b2, W3, b3):
    pad = EPAD - E
    src = jnp.concatenate(
        [edge_index[0].astype(jnp.int32), jnp.zeros((pad,), jnp.int32)]
    ).reshape(EPAD // CHUNK, CHUNK)
    dst = jnp.concatenate(
        [edge_index[1].astype(jnp.int32), jnp.full((pad,), N, jnp.int32)]
    ).reshape(EPAD // CHUNK, CHUNK)
    degp = _deg_kernel(dst).reshape(NC, N)
    ht1, dinv = _h1(features, W1, degp[0][:, None], degp[1][:, None])
    agg1 = _agg128(src, dst, ht1)
    ht2 = _fuse(agg1, ht1, dinv, b1[None, :], W2, 128, 128)
    agg2 = _agg128(src, dst, ht2)
    w3p = jnp.pad(W3, ((0, 0), (0, 64)))
    ht3 = _fuse(agg2, ht2, dinv, b2[None, :], w3p, 128, 128)
    agg3 = _agg128(src, dst, ht3)
    return _final(agg3, ht3, dinv, b3[None, :])
